# baseline (speedup 1.0000x reference)
import functools

import jax
import jax.numpy as jnp
from jax import lax
from jax.experimental import pallas as pl
from jax.experimental.pallas import tpu as pltpu
from jax.experimental.pallas import tpu_sc as plsc

_NC = 2
_NS = 16
_NW = _NC * _NS
_LANES = 16


def _project_table(emb, W, input_ids):
    V, D = emb.shape
    BK = 16384
    nblk = (V + BK - 1) // BK
    grid = (nblk + 1) // 2

    def body(emb0_ref, emb1_ref, w_ref, ids_ref, p0_ref, p1_ref, inv_ref):
        def mv(eref):
            row = lax.dot_general(
                w_ref[...], eref[...],
                (((1,), (1,)), ((), ())),
                preferred_element_type=jnp.float32,
            )
            return jnp.reshape(row, (BK,))

        p0_ref[...] = mv(emb0_ref)
        p1_ref[...] = mv(emb1_ref)

        @pl.when(pl.program_id(0) == 0)
        def _():
            cnt = jnp.sum((ids_ref[...] != 0).astype(jnp.float32))
            inv_ref[...] = jnp.full((1, 1), 1.0, jnp.float32) / cnt

    p0, p1, inv = pl.pallas_call(
        body,
        grid=(grid,),
        in_specs=[
            pl.BlockSpec((BK, D), lambda i: (i, 0)),
            pl.BlockSpec((BK, D),
                         lambda i: (jnp.minimum(i + grid, nblk - 1), 0)),
            pl.BlockSpec(W.shape, lambda i: (0, 0)),
            pl.BlockSpec(input_ids.shape, lambda i: (0, 0)),
        ],
        out_specs=[
            pl.BlockSpec((BK,), lambda i: (i,)),
            pl.BlockSpec((BK,), lambda i: (i,)),
            pl.BlockSpec((1, 1), lambda i: (0, 0)),
        ],
        out_shape=[
            jax.ShapeDtypeStruct((grid * BK,), jnp.float32),
            jax.ShapeDtypeStruct((grid * BK,), jnp.float32),
            jax.ShapeDtypeStruct((1, 1), jnp.float32),
        ],
    )(emb, emb, W, input_ids)
    return p0, p1, inv


def _make_sc_pool(B, L):
    rpw = B // _NW
    n_chunks = rpw // _LANES
    mesh = plsc.VectorSubcoreMesh(core_axis_name="c", subcore_axis_name="s")

    @functools.partial(
        pl.kernel,
        out_type=jax.ShapeDtypeStruct((B,), jnp.float32),
        mesh=mesh,
        scratch_types=[
            pltpu.VMEM((rpw * L,), jnp.int32),
            pltpu.VMEM((L, rpw), jnp.int32),
            pltpu.VMEM((L, rpw), jnp.float32),
            pltpu.VMEM((rpw,), jnp.float32),
            pltpu.VMEM((_LANES,), jnp.float32),
            pltpu.VMEM((_LANES,), jnp.float32),
            pltpu.SemaphoreType.DMA,
        ],
        compiler_params=pltpu.CompilerParams(
            use_tc_tiling_on_sc=False, needs_layout_passes=False),
    )
    def sc_pool(p_hbm, ids_hbm, inv_hbm, b_hbm, out_hbm,
                idx_n, idx_t, vals_v, out_v, inv_v, b_v, sem):
        wid = lax.axis_index("s") * _NC + lax.axis_index("c")
        base = wid * rpw
        pltpu.sync_copy(ids_hbm.at[pl.ds(base * L, rpw * L)], idx_n)
        pltpu.sync_copy(inv_hbm, inv_v)
        pltpu.sync_copy(b_hbm, b_v)

        for rc in range(n_chunks):
            flat0 = (jnp.full((_LANES,), rc * _LANES, jnp.int32) + lax.iota(
                jnp.int32, _LANES)) * L

            def tbody(t, carry, flat0=flat0, rc=rc):
                v = plsc.load_gather(idx_n, [flat0 + t])
                idx_t[t, pl.ds(rc * _LANES, _LANES)] = v
                return carry

            lax.fori_loop(0, L, tbody, 0)

        def fire(t, carry):
            pltpu.async_copy(p_hbm.at[idx_t.at[t]], vals_v.at[t], sem)
            return carry

        lax.fori_loop(0, L, fire, 0)

        def drain(t, carry):
            pltpu.make_async_copy(p_hbm.at[idx_t.at[t]], vals_v.at[t],
                                  sem).wait()
            return carry

        lax.fori_loop(0, L, drain, 0)

        inv = inv_v[...]
        bias = b_v[...]
        for rc in range(n_chunks):
            sl = pl.ds(rc * _LANES, _LANES)

            def body(t, acc, sl=sl):
                return acc + vals_v[t, sl]

            acc = lax.fori_loop(0, L, body, jnp.zeros((_LANES,), jnp.float32))
            out_v[sl] = acc * inv + bias
        pltpu.sync_copy(out_v, out_hbm.at[pl.ds(base, rpw)])

    return sc_pool


def _make_sc_bw_probe(V, D):
    rows_pw = 2048
    n_iter = V // (_NW * rows_pw)
    mesh = plsc.VectorSubcoreMesh(core_axis_name="c", subcore_axis_name="s")

    @functools.partial(
        pl.kernel,
        out_type=jax.ShapeDtypeStruct((_NW, _LANES), jnp.float32),
        mesh=mesh,
        scratch_types=[
            pltpu.VMEM((rows_pw, D), jnp.float32),
            pltpu.VMEM((_LANES,), jnp.float32),
        ],
        compiler_params=pltpu.CompilerParams(
            use_tc_tiling_on_sc=False, needs_layout_passes=False),
    )
    def bw_probe(emb_hbm, out_hbm, buf_v, o_v):
        wid = lax.axis_index("s") * _NC + lax.axis_index("c")
        base = wid * (rows_pw * n_iter)

        def body(k, carry):
            pltpu.sync_copy(
                emb_hbm.at[pl.ds(base + k * rows_pw, rows_pw), :], buf_v)
            return carry

        lax.fori_loop(0, n_iter, body, 0)
        o_v[...] = buf_v[0, pl.ds(0, _LANES)]
        pltpu.sync_copy(o_v, out_hbm.at[wid])

    return bw_probe


def kernel(input_ids, emb, W, b):
    B, L = input_ids.shape
    V, D = emb.shape
    probe = _make_sc_bw_probe(V, D)(emb)
    p0, p1, inv = _project_table(emb, W, input_ids)
    s = probe[:1, :1] + p0[:1, None] + p1[:1, None] + inv
    return jnp.broadcast_to(s, (B, 1))


def _unused_kernel(input_ids, emb, W, b):
    B, L = input_ids.shape
    V, D = emb.shape
    p0, p1, inv = _project_table(emb, W, input_ids)
    p = jnp.concatenate([p0, p1])[:V]
    inv16 = jnp.broadcast_to(jnp.reshape(inv, (1,)), (_LANES,))
    b16 = jnp.broadcast_to(b, (_LANES,))
    ids_flat = jnp.reshape(input_ids, (B * L,))
    acc = _make_sc_pool(B, L)(p, ids_flat, inv16, b16)
    return acc.reshape(B, 1)


# speedup vs baseline: 1.7263x; 1.7263x over previous
import functools

import jax
import jax.numpy as jnp
from jax import lax
from jax.experimental import pallas as pl
from jax.experimental.pallas import tpu as pltpu
from jax.experimental.pallas import tpu_sc as plsc

_NC = 2
_NS = 16
_NW = _NC * _NS
_LANES = 16


def _project_table(emb, W, input_ids):
    V, D = emb.shape
    BK = 32768
    grid = (V + BK - 1) // BK

    def body(emb_ref, w_ref, ids_ref, p_ref, inv_ref):
        row = lax.dot_general(
            w_ref[...], emb_ref[...],
            (((1,), (1,)), ((), ())),
            preferred_element_type=jnp.float32,
        )
        p_ref[...] = jnp.reshape(row, (BK,))

        @pl.when(pl.program_id(0) == 0)
        def _():
            cnt = jnp.sum((ids_ref[...] != 0).astype(jnp.float32))
            inv_ref[...] = jnp.full((1, 1), 1.0, jnp.float32) / cnt

    return pl.pallas_call(
        body,
        grid=(grid,),
        in_specs=[
            pl.BlockSpec((BK, D), lambda i: (i, 0)),
            pl.BlockSpec(W.shape, lambda i: (0, 0)),
            pl.BlockSpec(input_ids.shape, lambda i: (0, 0)),
        ],
        out_specs=[
            pl.BlockSpec((BK,), lambda i: (i,)),
            pl.BlockSpec((1, 1), lambda i: (0, 0)),
        ],
        out_shape=[
            jax.ShapeDtypeStruct((V,), jnp.float32),
            jax.ShapeDtypeStruct((1, 1), jnp.float32),
        ],
    )(emb, W, input_ids)


def _make_sc_pool(B, L):
    rpw = B // _NW
    n_chunks = rpw // _LANES
    mesh = plsc.VectorSubcoreMesh(core_axis_name="c", subcore_axis_name="s")

    @functools.partial(
        pl.kernel,
        out_type=jax.ShapeDtypeStruct((B,), jnp.float32),
        mesh=mesh,
        scratch_types=[
            pltpu.VMEM((rpw * L,), jnp.int32),
            pltpu.VMEM((L, rpw), jnp.int32),
            pltpu.VMEM((L, rpw), jnp.float32),
            pltpu.VMEM((rpw,), jnp.float32),
            pltpu.VMEM((_LANES,), jnp.float32),
            pltpu.VMEM((_LANES,), jnp.float32),
            pltpu.SemaphoreType.DMA,
        ],
        compiler_params=pltpu.CompilerParams(
            use_tc_tiling_on_sc=False, needs_layout_passes=False),
    )
    def sc_pool(p_hbm, ids_hbm, inv_hbm, b_hbm, out_hbm,
                idx_n, idx_t, vals_v, out_v, inv_v, b_v, sem):
        wid = lax.axis_index("s") * _NC + lax.axis_index("c")
        base = wid * rpw
        pltpu.sync_copy(ids_hbm.at[pl.ds(base * L, rpw * L)], idx_n)
        pltpu.sync_copy(inv_hbm, inv_v)
        pltpu.sync_copy(b_hbm, b_v)

        for rc in range(n_chunks):
            flat0 = (jnp.full((_LANES,), rc * _LANES, jnp.int32) + lax.iota(
                jnp.int32, _LANES)) * L

            def tbody(t, carry, flat0=flat0):
                v = plsc.load_gather(idx_n, [flat0 + t])
                idx_t[t, pl.ds(rc * _LANES, _LANES)] = v
                return carry

            lax.fori_loop(0, L, tbody, 0)

        def fire(t, carry):
            pltpu.async_copy(p_hbm.at[idx_t.at[t]], vals_v.at[t], sem)
            return carry

        lax.fori_loop(0, L, fire, 0)

        def drain(t, carry):
            pltpu.make_async_copy(p_hbm.at[idx_t.at[t]], vals_v.at[t],
                                  sem).wait()
            return carry

        lax.fori_loop(0, L, drain, 0)

        inv = inv_v[...]
        bias = b_v[...]
        for rc in range(n_chunks):
            sl = pl.ds(rc * _LANES, _LANES)

            def body(t, acc, sl=sl):
                return acc + vals_v[t, sl]

            acc = lax.fori_loop(0, L, body, jnp.zeros((_LANES,), jnp.float32))
            out_v[sl] = acc * inv + bias
        pltpu.sync_copy(out_v, out_hbm.at[pl.ds(base, rpw)])

    return sc_pool


def kernel(input_ids, emb, W, b):
    B, L = input_ids.shape
    p, inv = _project_table(emb, W, input_ids)
    inv16 = jnp.broadcast_to(jnp.reshape(inv, (1,)), (_LANES,))
    b16 = jnp.broadcast_to(b, (_LANES,))
    ids_flat = jnp.reshape(input_ids, (B * L,))
    acc = _make_sc_pool(B, L)(p, ids_flat, inv16, b16)
    return acc.reshape(B, 1)
